# baseline (speedup 1.0000x reference)
import jax
import jax.numpy as jnp
from jax import lax
from jax.experimental import pallas as pl
from jax.experimental.pallas import tpu as pltpu
from jax.experimental.pallas import tpu_sc as plsc

N = 10000
E = 320000
DIM = 128
NCLS = 16
ALPHA = 0.2

NW = 32
NP = 10240
EW = E // NW
CH = 128
NCH = 80
EPAD = NCH * CH
SINK = NP - 1
RPW = NP // 16
RB = N // 10

_f32 = jnp.float32
_i32 = jnp.int32

_BCAST_DNUMS = lax.GatherDimensionNumbers(
    offset_dims=(), collapsed_slice_dims=(0,), start_index_map=(0,))


def _lane_bcast(v, r):
    return lax.gather(v, jnp.full((16, 1), r, _i32), _BCAST_DNUMS,
                      slice_sizes=(1,),
                      mode=lax.GatherScatterMode.PROMISE_IN_BOUNDS)


def _prep_body(x_ref, w_ref, am_ref, h_ref, s12_ref):
    xb = x_ref[...]
    hb = jnp.dot(xb, w_ref[...], preferred_element_type=_f32)
    h_ref[...] = hb
    s12_ref[...] = jnp.dot(hb, am_ref[...], preferred_element_type=_f32)


def _prep(x, W, amat):
    return pl.pallas_call(
        _prep_body,
        grid=(N // RB,),
        in_specs=[
            pl.BlockSpec((RB, DIM), lambda i: (i, 0)),
            pl.BlockSpec((DIM, NCLS), lambda i: (0, 0)),
            pl.BlockSpec((NCLS, 2), lambda i: (0, 0)),
        ],
        out_specs=[
            pl.BlockSpec((RB, NCLS), lambda i: (i, 0)),
            pl.BlockSpec((RB, 2), lambda i: (i, 0)),
        ],
        out_shape=[
            jax.ShapeDtypeStruct((N, NCLS), _f32),
            jax.ShapeDtypeStruct((N, 2), _f32),
        ],
    )(x, W, amat)


def _sc_body(h_hbm, s12_hbm, src_hbm, dst_hbm,
             num_out, den_out,
             srcb, dstb, s12b, rows0, rows1, dval0, dval1, zbuf, dzbuf,
             num_sh, den_sh, gsem, ssem):
    cid = lax.axis_index("c")
    sid = lax.axis_index("s")
    wid = sid * 2 + cid

    def _z(i, carry):
        zbuf[i, :] = jnp.zeros((16,), _f32)
        return carry
    lax.fori_loop(0, RPW, _z, 0)
    for i in range(RPW // 16):
        dzbuf[pl.ds(i * 16, 16)] = jnp.zeros((16,), _f32)
    pltpu.sync_copy(dzbuf, den_sh.at[pl.ds(sid * RPW, RPW)])
    pltpu.sync_copy(zbuf, num_sh.at[pl.ds(sid * RPW, RPW)])

    pltpu.sync_copy(src_hbm.at[wid], srcb)
    pltpu.sync_copy(dst_hbm.at[wid], dstb)
    pltpu.sync_copy(s12_hbm, s12b)

    plsc.subcore_barrier()

    zeros16 = jnp.zeros((16,), _i32)
    ones16 = jnp.ones((16,), _i32)

    pltpu.async_copy(h_hbm.at[dstb.at[0]], rows0, gsem)

    def _pair(j2, carry):
        for b in range(2):
            j = 2 * j2 + b
            rbuf, obuf = (rows0, rows1) if b == 0 else (rows1, rows0)
            dbuf, odbuf = (dval0, dval1) if b == 0 else (dval1, dval0)

            pltpu.make_async_copy(h_hbm.at[dstb.at[j]], rbuf, gsem).wait()

            @pl.when(j >= 1)
            def _wait_prev():
                pltpu.make_async_copy(
                    obuf, num_sh.at[srcb.at[j - 1]], ssem).wait()
                pltpu.make_async_copy(
                    odbuf, den_sh.at[srcb.at[j - 1]], ssem).wait()

            @pl.when(j < NCH - 1)
            def _prefetch():
                pltpu.async_copy(h_hbm.at[dstb.at[j + 1]], obuf, gsem)

            for k in range(CH // 16):
                src16 = srcb[j, pl.ds(k * 16, 16)]
                dst16 = dstb[j, pl.ds(k * 16, 16)]
                sv = plsc.load_gather(s12b, [src16, zeros16])
                dv = plsc.load_gather(s12b, [dst16, ones16])
                t = sv + dv
                lr = jnp.where(t > 0, t, ALPHA * t)
                e = jnp.exp(-lr)
                dbuf[pl.ds(k * 16, 16)] = e
                for r in range(16):
                    idx = k * 16 + r
                    ev = _lane_bcast(e, r)
                    rbuf[idx, :] = rbuf[idx, :] * ev
            pltpu.async_copy(rbuf, num_sh.at[srcb.at[j]], ssem, add=True)
            pltpu.async_copy(dbuf, den_sh.at[srcb.at[j]], ssem, add=True)
        return carry

    lax.fori_loop(0, NCH // 2, _pair, 0)
    pltpu.make_async_copy(rows1, num_sh.at[srcb.at[NCH - 1]], ssem).wait()
    pltpu.make_async_copy(dval1, den_sh.at[srcb.at[NCH - 1]], ssem).wait()

    plsc.subcore_barrier()

    sl = pl.ds(sid * RPW, RPW)
    pltpu.sync_copy(num_sh.at[sl], num_out.at[cid, sl])
    pltpu.sync_copy(den_sh.at[sl], den_out.at[cid, sl])


def _sc_edges(h, s12, srcp, dstp):
    mesh = plsc.VectorSubcoreMesh(core_axis_name="c", subcore_axis_name="s")
    f = pl.kernel(
        _sc_body,
        out_type=[
            jax.ShapeDtypeStruct((2, NP, NCLS), _f32),
            jax.ShapeDtypeStruct((2, NP), _f32),
        ],
        mesh=mesh,
        compiler_params=pltpu.CompilerParams(
            needs_layout_passes=False, use_tc_tiling_on_sc=False),
        scratch_types=[
            pltpu.VMEM((NCH, CH), _i32),
            pltpu.VMEM((NCH, CH), _i32),
            pltpu.VMEM((N, 2), _f32),
            pltpu.VMEM((CH, NCLS), _f32),
            pltpu.VMEM((CH, NCLS), _f32),
            pltpu.VMEM((CH,), _f32),
            pltpu.VMEM((CH,), _f32),
            pltpu.VMEM((RPW, 16), _f32),
            pltpu.VMEM((RPW,), _f32),
            pltpu.VMEM_SHARED((NP, NCLS), _f32),
            pltpu.VMEM_SHARED((NP,), _f32),
            pltpu.SemaphoreType.DMA,
            pltpu.SemaphoreType.DMA,
        ],
    )
    return f(h, s12, srcp, dstp)


def _fin_body(np_ref, dp_ref, out_ref):
    nm = np_ref[0] + np_ref[1]
    dn = dp_ref[0] + dp_ref[1] + 1e-16
    r = nm / dn
    out_ref[...] = jnp.where(r > 0, r, jnp.exp(r) - 1.0)


def _finalize(num_p, den_p3):
    return pl.pallas_call(
        _fin_body,
        grid=(N // RB,),
        in_specs=[
            pl.BlockSpec((2, RB, NCLS), lambda i: (0, i, 0)),
            pl.BlockSpec((2, RB, 1), lambda i: (0, i, 0)),
        ],
        out_specs=pl.BlockSpec((RB, NCLS), lambda i: (i, 0)),
        out_shape=jax.ShapeDtypeStruct((N, NCLS), _f32),
    )(num_p, den_p3)


def kernel(x, adj, W, a):
    amat = jnp.transpose(a.reshape(2, NCLS))
    src = adj[0].reshape(NW, EW)
    dst = adj[1].reshape(NW, EW)
    srcp = jnp.concatenate(
        [src, jnp.full((NW, EPAD - EW), SINK, _i32)], axis=1
    ).reshape(NW, NCH, CH)
    dstp = jnp.concatenate(
        [dst, jnp.zeros((NW, EPAD - EW), _i32)], axis=1
    ).reshape(NW, NCH, CH)

    h, s12 = _prep(x, W, amat)
    num_p, den_p = _sc_edges(h, s12, srcp, dstp)
    return _finalize(num_p, den_p.reshape(2, NP, 1))


# speedup vs baseline: 1.1495x; 1.1495x over previous
import jax
import jax.numpy as jnp
from jax import lax
from jax.experimental import pallas as pl
from jax.experimental.pallas import tpu as pltpu
from jax.experimental.pallas import tpu_sc as plsc

N = 10000
E = 320000
DIM = 128
NCLS = 16
ALPHA = 0.2

NW = 32
NP = 10240
EW = E // NW
CH = 128
NCH = 80
EPAD = NCH * CH
SINK = NP - 1
RPW = NP // 16
RB = N // 10
NB = 8

_f32 = jnp.float32
_i32 = jnp.int32

_BCAST_DNUMS = lax.GatherDimensionNumbers(
    offset_dims=(), collapsed_slice_dims=(0,), start_index_map=(0,))


def _lane_bcast(v, r, zeros16):
    idx = jnp.reshape(zeros16 + jnp.int32(r), (16, 1))
    return lax.gather(v, idx, _BCAST_DNUMS, slice_sizes=(1,),
                      mode=lax.GatherScatterMode.PROMISE_IN_BOUNDS)


def _prep_body(x_ref, w_ref, am_ref, h_ref, s12_ref):
    xb = x_ref[...]
    hb = jnp.dot(xb, w_ref[...], preferred_element_type=_f32)
    h_ref[...] = hb
    s12_ref[...] = jnp.dot(hb, am_ref[...], preferred_element_type=_f32)


def _prep(x, W, amat):
    return pl.pallas_call(
        _prep_body,
        grid=(N // RB,),
        in_specs=[
            pl.BlockSpec((RB, DIM), lambda i: (i, 0)),
            pl.BlockSpec((DIM, NCLS), lambda i: (0, 0)),
            pl.BlockSpec((NCLS, 2), lambda i: (0, 0)),
        ],
        out_specs=[
            pl.BlockSpec((RB, NCLS), lambda i: (i, 0)),
            pl.BlockSpec((RB, 2), lambda i: (i, 0)),
        ],
        out_shape=[
            jax.ShapeDtypeStruct((N, NCLS), _f32),
            jax.ShapeDtypeStruct((N, 2), _f32),
        ],
    )(x, W, amat)


def _sc_body(h_hbm, s12_hbm, src_hbm, dst_hbm,
             num_out, den_out,
             srcb, dstb, s12b, rows, dvals,
             num_sh, den_sh, gsem, ssem):
    cid = lax.axis_index("c")
    sid = lax.axis_index("s")
    wid = sid * 2 + cid

    def _z(i, carry):
        for b in range(NB):
            rows[b, i, :] = jnp.zeros((16,), _f32)
        return carry
    lax.fori_loop(0, CH, _z, 0)
    for b in range(NB):
        for i in range(CH // 16):
            dvals[b, pl.ds(i * 16, 16)] = jnp.zeros((16,), _f32)
    for b in range(RPW // CH):
        pltpu.sync_copy(
            rows.at[b], num_sh.at[pl.ds(sid * RPW + b * CH, CH)])
        pltpu.sync_copy(
            dvals.at[b], den_sh.at[pl.ds(sid * RPW + b * CH, CH)])

    pltpu.sync_copy(src_hbm.at[wid], srcb)
    pltpu.sync_copy(dst_hbm.at[wid], dstb)
    pltpu.sync_copy(s12_hbm, s12b)

    plsc.subcore_barrier()

    zeros16 = jnp.zeros((16,), _i32)
    ones16 = jnp.ones((16,), _i32)

    for b in range(4):
        pltpu.async_copy(h_hbm.at[dstb.at[b]], rows.at[b], gsem.at[b])

    def _oct(jj, carry):
        for b in range(NB):
            j = NB * jj + b
            rbuf = rows.at[b]
            dbuf = dvals.at[b]
            b2 = (b + 4) % NB

            pltpu.make_async_copy(
                h_hbm.at[dstb.at[j]], rbuf, gsem.at[b]).wait()

            @pl.when(j >= 4)
            def _wait_prev():
                pltpu.make_async_copy(
                    rows.at[b2], num_sh.at[srcb.at[j - 4]],
                    ssem.at[b2]).wait()
                pltpu.make_async_copy(
                    dvals.at[b2], den_sh.at[srcb.at[j - 4]],
                    ssem.at[b2]).wait()

            @pl.when(j + 4 < NCH)
            def _prefetch():
                pltpu.async_copy(
                    h_hbm.at[dstb.at[j + 4]], rows.at[b2], gsem.at[b2])

            for k in range(CH // 16):
                src16 = srcb[j, pl.ds(k * 16, 16)]
                dst16 = dstb[j, pl.ds(k * 16, 16)]
                sv = plsc.load_gather(s12b, [src16, zeros16])
                dv = plsc.load_gather(s12b, [dst16, ones16])
                t = sv + dv
                lr = jnp.where(t > 0, t, ALPHA * t)
                e = jnp.exp(-lr)
                dbuf[pl.ds(k * 16, 16)] = e
                for r in range(16):
                    idx = k * 16 + r
                    ev = _lane_bcast(e, r, zeros16)
                    rbuf[idx, :] = rbuf[idx, :] * ev
            pltpu.async_copy(rbuf, num_sh.at[srcb.at[j]], ssem.at[b],
                             add=True)
            pltpu.async_copy(dbuf, den_sh.at[srcb.at[j]], ssem.at[b],
                             add=True)
        return carry

    lax.fori_loop(0, NCH // NB, _oct, 0)
    for b in range(4, NB):
        j = NCH - NB + b
        pltpu.make_async_copy(
            rows.at[b], num_sh.at[srcb.at[j]], ssem.at[b]).wait()
        pltpu.make_async_copy(
            dvals.at[b], den_sh.at[srcb.at[j]], ssem.at[b]).wait()

    plsc.subcore_barrier()

    sl = pl.ds(sid * RPW, RPW)
    pltpu.sync_copy(num_sh.at[sl], num_out.at[cid, sl])
    pltpu.sync_copy(den_sh.at[sl], den_out.at[cid, sl])


def _sc_edges(h, s12, srcp, dstp):
    mesh = plsc.VectorSubcoreMesh(core_axis_name="c", subcore_axis_name="s")
    f = pl.kernel(
        _sc_body,
        out_type=[
            jax.ShapeDtypeStruct((2, NP, NCLS), _f32),
            jax.ShapeDtypeStruct((2, NP), _f32),
        ],
        mesh=mesh,
        compiler_params=pltpu.CompilerParams(
            needs_layout_passes=False, use_tc_tiling_on_sc=False),
        scratch_types=[
            pltpu.VMEM((NCH, CH), _i32),
            pltpu.VMEM((NCH, CH), _i32),
            pltpu.VMEM((N, 2), _f32),
            pltpu.VMEM((NB, CH, NCLS), _f32),
            pltpu.VMEM((NB, CH), _f32),
            pltpu.VMEM_SHARED((NP, NCLS), _f32),
            pltpu.VMEM_SHARED((NP,), _f32),
            pltpu.SemaphoreType.DMA((NB,)),
            pltpu.SemaphoreType.DMA((NB,)),
        ],
    )
    return f(h, s12, srcp, dstp)


def _fin_body(np_ref, dp_ref, out_ref):
    nm = np_ref[0] + np_ref[1]
    dn = dp_ref[0] + dp_ref[1] + 1e-16
    r = nm / dn
    out_ref[...] = jnp.where(r > 0, r, jnp.exp(r) - 1.0)


def _finalize(num_p, den_p3):
    return pl.pallas_call(
        _fin_body,
        grid=(N // RB,),
        in_specs=[
            pl.BlockSpec((2, RB, NCLS), lambda i: (0, i, 0)),
            pl.BlockSpec((2, RB, 1), lambda i: (0, i, 0)),
        ],
        out_specs=pl.BlockSpec((RB, NCLS), lambda i: (i, 0)),
        out_shape=jax.ShapeDtypeStruct((N, NCLS), _f32),
    )(num_p, den_p3)


def kernel(x, adj, W, a):
    amat = jnp.transpose(a.reshape(2, NCLS))
    src = adj[0].reshape(NW, EW)
    dst = adj[1].reshape(NW, EW)
    srcp = jnp.concatenate(
        [src, jnp.full((NW, EPAD - EW), SINK, _i32)], axis=1
    ).reshape(NW, NCH, CH)
    dstp = jnp.concatenate(
        [dst, jnp.zeros((NW, EPAD - EW), _i32)], axis=1
    ).reshape(NW, NCH, CH)

    h, s12 = _prep(x, W, amat)
    num_p, den_p = _sc_edges(h, s12, srcp, dstp)
    return _finalize(num_p, den_p.reshape(2, NP, 1))


# speedup vs baseline: 1.4392x; 1.2520x over previous
import jax
import jax.numpy as jnp
from jax import lax
from jax.experimental import pallas as pl
from jax.experimental.pallas import tpu as pltpu
from jax.experimental.pallas import tpu_sc as plsc

N = 10000
E = 320000
DIM = 128
NCLS = 16
ALPHA = 0.2

NW = 32
NP = 10240
EW = E // NW
CH = 128
NCH = 80
EPAD = NCH * CH
SINK = NP - 1
RPW = NP // 16
RB = N // 10
NB = 8

_f32 = jnp.float32
_i32 = jnp.int32

_BCAST_DNUMS = lax.GatherDimensionNumbers(
    offset_dims=(), collapsed_slice_dims=(0,), start_index_map=(0,))


def _lane_bcast(v, r, zeros16):
    idx = jnp.reshape(zeros16 + jnp.int32(r), (16, 1))
    return lax.gather(v, idx, _BCAST_DNUMS, slice_sizes=(1,),
                      mode=lax.GatherScatterMode.PROMISE_IN_BOUNDS)


def _prep_body(x_ref, w_ref, am_ref, h_ref, s12_ref):
    xb = x_ref[...]
    hb = jnp.dot(xb, w_ref[...], preferred_element_type=_f32)
    h_ref[...] = hb
    s12_ref[...] = jnp.dot(hb, am_ref[...], preferred_element_type=_f32)


def _prep(x, W, amat):
    return pl.pallas_call(
        _prep_body,
        grid=(N // RB,),
        in_specs=[
            pl.BlockSpec((RB, DIM), lambda i: (i, 0)),
            pl.BlockSpec((DIM, NCLS), lambda i: (0, 0)),
            pl.BlockSpec((NCLS, 2), lambda i: (0, 0)),
        ],
        out_specs=[
            pl.BlockSpec((RB, NCLS), lambda i: (i, 0)),
            pl.BlockSpec((RB, 2), lambda i: (i, 0)),
        ],
        out_shape=[
            jax.ShapeDtypeStruct((N, NCLS), _f32),
            jax.ShapeDtypeStruct((N, 2), _f32),
        ],
    )(x, W, amat)


def _sc_body(h_hbm, s12_hbm, adj_hbm,
             num_out, den_out,
             srcb, dstb, s12b, rows, dvals,
             num_sh, den_sh, gsem, ssem):
    cid = lax.axis_index("c")
    sid = lax.axis_index("s")
    wid = sid * 2 + cid
    estart = wid * EW
    r0 = jnp.minimum(estart // CH, E // CH - NCH)
    off = estart - r0 * CH

    def _z(i, carry):
        for b in range(NB):
            rows[b, i, :] = jnp.zeros((16,), _f32)
        return carry
    lax.fori_loop(0, CH, _z, 0)
    for b in range(NB):
        for i in range(CH // 16):
            dvals[b, pl.ds(i * 16, 16)] = jnp.zeros((16,), _f32)
    for b in range(RPW // CH):
        pltpu.sync_copy(
            rows.at[b], num_sh.at[pl.ds(sid * RPW + b * CH, CH)])
        pltpu.sync_copy(
            dvals.at[b], den_sh.at[pl.ds(sid * RPW + b * CH, CH)])

    pltpu.sync_copy(adj_hbm.at[0, pl.ds(r0, NCH)], srcb)
    pltpu.sync_copy(adj_hbm.at[1, pl.ds(r0, NCH)], dstb)
    pltpu.sync_copy(s12_hbm, s12b)

    plsc.subcore_barrier()

    zeros16 = jnp.zeros((16,), _i32)
    ones16 = jnp.ones((16,), _i32)
    iota16 = lax.iota(_i32, 16)

    for b in range(4):
        pltpu.async_copy(h_hbm.at[dstb.at[b]], rows.at[b], gsem.at[b])

    def _oct(jj, carry):
        for b in range(NB):
            j = NB * jj + b
            rbuf = rows.at[b]
            dbuf = dvals.at[b]
            b2 = (b + 4) % NB

            pltpu.make_async_copy(
                h_hbm.at[dstb.at[j]], rbuf, gsem.at[b]).wait()

            @pl.when(j >= 4)
            def _wait_prev():
                pltpu.make_async_copy(
                    rows.at[b2], num_sh.at[srcb.at[j - 4]],
                    ssem.at[b2]).wait()
                pltpu.make_async_copy(
                    dvals.at[b2], den_sh.at[srcb.at[j - 4]],
                    ssem.at[b2]).wait()

            @pl.when(j + 4 < NCH)
            def _prefetch():
                pltpu.async_copy(
                    h_hbm.at[dstb.at[j + 4]], rows.at[b2], gsem.at[b2])

            for k in range(CH // 16):
                src16 = srcb[j, pl.ds(k * 16, 16)]
                dst16 = dstb[j, pl.ds(k * 16, 16)]
                sv = plsc.load_gather(s12b, [src16, zeros16])
                dv = plsc.load_gather(s12b, [dst16, ones16])
                t = sv + dv
                lr = jnp.where(t > 0, t, ALPHA * t)
                e = jnp.exp(-lr)
                flat = iota16 + (j * CH + k * 16)
                e = jnp.where((flat >= off) & (flat < off + EW), e, 0.0)
                dbuf[pl.ds(k * 16, 16)] = e
                for r in range(16):
                    idx = k * 16 + r
                    ev = _lane_bcast(e, r, zeros16)
                    rbuf[idx, :] = rbuf[idx, :] * ev
            pltpu.async_copy(rbuf, num_sh.at[srcb.at[j]], ssem.at[b],
                             add=True)
            pltpu.async_copy(dbuf, den_sh.at[srcb.at[j]], ssem.at[b],
                             add=True)
        return carry

    lax.fori_loop(0, NCH // NB, _oct, 0)
    for b in range(4, NB):
        j = NCH - NB + b
        pltpu.make_async_copy(
            rows.at[b], num_sh.at[srcb.at[j]], ssem.at[b]).wait()
        pltpu.make_async_copy(
            dvals.at[b], den_sh.at[srcb.at[j]], ssem.at[b]).wait()

    plsc.subcore_barrier()

    sl = pl.ds(sid * RPW, RPW)
    pltpu.sync_copy(num_sh.at[sl], num_out.at[cid, sl])
    pltpu.sync_copy(den_sh.at[sl], den_out.at[cid, sl])


def _sc_edges(h, s12, adj):
    mesh = plsc.VectorSubcoreMesh(core_axis_name="c", subcore_axis_name="s")
    f = pl.kernel(
        _sc_body,
        out_type=[
            jax.ShapeDtypeStruct((2, NP, NCLS), _f32),
            jax.ShapeDtypeStruct((2, NP), _f32),
        ],
        mesh=mesh,
        compiler_params=pltpu.CompilerParams(
            needs_layout_passes=False, use_tc_tiling_on_sc=False),
        scratch_types=[
            pltpu.VMEM((NCH, CH), _i32),
            pltpu.VMEM((NCH, CH), _i32),
            pltpu.VMEM((N, 2), _f32),
            pltpu.VMEM((NB, CH, NCLS), _f32),
            pltpu.VMEM((NB, CH), _f32),
            pltpu.VMEM_SHARED((NP, NCLS), _f32),
            pltpu.VMEM_SHARED((NP,), _f32),
            pltpu.SemaphoreType.DMA((NB,)),
            pltpu.SemaphoreType.DMA((NB,)),
        ],
    )
    return f(h, s12, adj)


def _fin_body(np_ref, dp_ref, out_ref):
    nm = np_ref[0] + np_ref[1]
    dn = dp_ref[0] + dp_ref[1] + 1e-16
    r = nm / dn
    out_ref[...] = jnp.where(r > 0, r, jnp.exp(r) - 1.0)


def _finalize(num_p, den_p3):
    return pl.pallas_call(
        _fin_body,
        grid=(N // RB,),
        in_specs=[
            pl.BlockSpec((2, RB, NCLS), lambda i: (0, i, 0)),
            pl.BlockSpec((2, RB, 1), lambda i: (0, i, 0)),
        ],
        out_specs=pl.BlockSpec((RB, NCLS), lambda i: (i, 0)),
        out_shape=jax.ShapeDtypeStruct((N, NCLS), _f32),
    )(num_p, den_p3)


def kernel(x, adj, W, a):
    amat = jnp.transpose(a.reshape(2, NCLS))
    h, s12 = _prep(x, W, amat)
    num_p, den_p = _sc_edges(h, s12, adj.reshape(2, E // CH, CH))
    return _finalize(num_p, den_p.reshape(2, NP, 1))


# speedup vs baseline: 1.4421x; 1.0021x over previous
import jax
import jax.numpy as jnp
from jax import lax
from jax.experimental import pallas as pl
from jax.experimental.pallas import tpu as pltpu
from jax.experimental.pallas import tpu_sc as plsc

N = 10000
E = 320000
DIM = 128
NCLS = 16
ALPHA = 0.2

NW = 32
NP = 10240
EW = E // NW
CH = 128
NCH = 80
EPAD = NCH * CH
SINK = NP - 1
RPW = NP // 16
RB = N // 10
NB = 8

_f32 = jnp.float32
_i32 = jnp.int32

_BCAST_DNUMS = lax.GatherDimensionNumbers(
    offset_dims=(), collapsed_slice_dims=(0,), start_index_map=(0,))


def _lane_bcast(v, r, zeros16):
    idx = jnp.reshape(zeros16 + jnp.int32(r), (16, 1))
    return lax.gather(v, idx, _BCAST_DNUMS, slice_sizes=(1,),
                      mode=lax.GatherScatterMode.PROMISE_IN_BOUNDS)


def _prep_body(x_ref, w_ref, am_ref, h_ref, s12_ref):
    xb = x_ref[...]
    hb = jnp.dot(xb, w_ref[...], preferred_element_type=_f32)
    h_ref[...] = hb
    s12_ref[...] = jnp.dot(hb, am_ref[...], preferred_element_type=_f32)


def _prep(x, W, amat):
    return pl.pallas_call(
        _prep_body,
        grid=(N // RB,),
        in_specs=[
            pl.BlockSpec((RB, DIM), lambda i: (i, 0)),
            pl.BlockSpec((DIM, NCLS), lambda i: (0, 0)),
            pl.BlockSpec((NCLS, 2), lambda i: (0, 0)),
        ],
        out_specs=[
            pl.BlockSpec((RB, NCLS), lambda i: (i, 0)),
            pl.BlockSpec((RB, 2), lambda i: (i, 0)),
        ],
        out_shape=[
            jax.ShapeDtypeStruct((N, NCLS), _f32),
            jax.ShapeDtypeStruct((N, 2), _f32),
        ],
    )(x, W, amat)


def _sc_body(h_hbm, s12_hbm, adj_hbm,
             num_out, den_out,
             srcb, dstb, s12b, rows, dvals,
             num_sh, den_sh, gsem, ssem):
    cid = lax.axis_index("c")
    sid = lax.axis_index("s")
    wid = sid * 2 + cid
    estart = wid * EW
    r0 = jnp.minimum(estart // CH, E // CH - NCH)
    off = estart - r0 * CH

    def _z(i, carry):
        for b in range(NB):
            rows[b, i, :] = jnp.zeros((16,), _f32)
        return carry
    lax.fori_loop(0, CH, _z, 0)
    for b in range(NB):
        for i in range(CH // 16):
            dvals[b, pl.ds(i * 16, 16)] = jnp.zeros((16,), _f32)
    for b in range(RPW // CH):
        pltpu.sync_copy(
            rows.at[b], num_sh.at[pl.ds(sid * RPW + b * CH, CH)])
        pltpu.sync_copy(
            dvals.at[b], den_sh.at[pl.ds(sid * RPW + b * CH, CH)])

    pltpu.sync_copy(adj_hbm.at[0, pl.ds(r0, NCH)], srcb)
    pltpu.sync_copy(adj_hbm.at[1, pl.ds(r0, NCH)], dstb)
    pltpu.sync_copy(s12_hbm, s12b)

    plsc.subcore_barrier()

    zeros16 = jnp.zeros((16,), _i32)
    ones16 = jnp.ones((16,), _i32)
    iota16 = lax.iota(_i32, 16)

    for b in range(6):
        pltpu.async_copy(h_hbm.at[dstb.at[b]], rows.at[b], gsem.at[b])

    def _oct(jj, carry):
        for b in range(NB):
            j = NB * jj + b
            rbuf = rows.at[b]
            dbuf = dvals.at[b]
            b2 = (b + 6) % NB

            pltpu.make_async_copy(
                h_hbm.at[dstb.at[j]], rbuf, gsem.at[b]).wait()

            @pl.when(j >= 2)
            def _wait_prev():
                pltpu.make_async_copy(
                    rows.at[b2], num_sh.at[srcb.at[j - 2]],
                    ssem.at[b2]).wait()
                pltpu.make_async_copy(
                    dvals.at[b2], den_sh.at[srcb.at[j - 2]],
                    ssem.at[b2]).wait()

            @pl.when(j + 6 < NCH)
            def _prefetch():
                pltpu.async_copy(
                    h_hbm.at[dstb.at[j + 6]], rows.at[b2], gsem.at[b2])

            for k in range(CH // 16):
                src16 = srcb[j, pl.ds(k * 16, 16)]
                dst16 = dstb[j, pl.ds(k * 16, 16)]
                sv = plsc.load_gather(s12b, [src16, zeros16])
                dv = plsc.load_gather(s12b, [dst16, ones16])
                t = sv + dv
                lr = jnp.where(t > 0, t, ALPHA * t)
                e = jnp.exp(-lr)
                flat = iota16 + (j * CH + k * 16)
                e = jnp.where((flat >= off) & (flat < off + EW), e, 0.0)
                dbuf[pl.ds(k * 16, 16)] = e
                for r in range(16):
                    idx = k * 16 + r
                    ev = _lane_bcast(e, r, zeros16)
                    rbuf[idx, :] = rbuf[idx, :] * ev
            pltpu.async_copy(rbuf, num_sh.at[srcb.at[j]], ssem.at[b],
                             add=True)
            pltpu.async_copy(dbuf, den_sh.at[srcb.at[j]], ssem.at[b],
                             add=True)
        return carry

    lax.fori_loop(0, NCH // NB, _oct, 0)
    for b in range(6, NB):
        j = NCH - NB + b
        pltpu.make_async_copy(
            rows.at[b], num_sh.at[srcb.at[j]], ssem.at[b]).wait()
        pltpu.make_async_copy(
            dvals.at[b], den_sh.at[srcb.at[j]], ssem.at[b]).wait()

    plsc.subcore_barrier()

    sl = pl.ds(sid * RPW, RPW)
    pltpu.sync_copy(num_sh.at[sl], num_out.at[cid, sl])
    pltpu.sync_copy(den_sh.at[sl], den_out.at[cid, sl])


def _sc_edges(h, s12, adj):
    mesh = plsc.VectorSubcoreMesh(core_axis_name="c", subcore_axis_name="s")
    f = pl.kernel(
        _sc_body,
        out_type=[
            jax.ShapeDtypeStruct((2, NP, NCLS), _f32),
            jax.ShapeDtypeStruct((2, NP), _f32),
        ],
        mesh=mesh,
        compiler_params=pltpu.CompilerParams(
            needs_layout_passes=False, use_tc_tiling_on_sc=False),
        scratch_types=[
            pltpu.VMEM((NCH, CH), _i32),
            pltpu.VMEM((NCH, CH), _i32),
            pltpu.VMEM((N, 2), _f32),
            pltpu.VMEM((NB, CH, NCLS), _f32),
            pltpu.VMEM((NB, CH), _f32),
            pltpu.VMEM_SHARED((NP, NCLS), _f32),
            pltpu.VMEM_SHARED((NP,), _f32),
            pltpu.SemaphoreType.DMA((NB,)),
            pltpu.SemaphoreType.DMA((NB,)),
        ],
    )
    return f(h, s12, adj)


def _fin_body(np_ref, dp_ref, out_ref):
    nm = np_ref[0] + np_ref[1]
    dn = dp_ref[0] + dp_ref[1] + 1e-16
    r = nm / dn
    out_ref[...] = jnp.where(r > 0, r, jnp.exp(r) - 1.0)


def _finalize(num_p, den_p3):
    return pl.pallas_call(
        _fin_body,
        grid=(N // RB,),
        in_specs=[
            pl.BlockSpec((2, RB, NCLS), lambda i: (0, i, 0)),
            pl.BlockSpec((2, RB, 1), lambda i: (0, i, 0)),
        ],
        out_specs=pl.BlockSpec((RB, NCLS), lambda i: (i, 0)),
        out_shape=jax.ShapeDtypeStruct((N, NCLS), _f32),
    )(num_p, den_p3)


def kernel(x, adj, W, a):
    amat = jnp.transpose(a.reshape(2, NCLS))
    h, s12 = _prep(x, W, amat)
    num_p, den_p = _sc_edges(h, s12, adj.reshape(2, E // CH, CH))
    return _finalize(num_p, den_p.reshape(2, NP, 1))


# speedup vs baseline: 1.4994x; 1.0397x over previous
import jax
import jax.numpy as jnp
from jax import lax
from jax.experimental import pallas as pl
from jax.experimental.pallas import tpu as pltpu
from jax.experimental.pallas import tpu_sc as plsc

N = 10000
E = 320000
DIM = 128
NCLS = 16
ALPHA = 0.2

NW = 32
NP = 10240
EW = E // NW
CH = 128
NCH = 80
EPAD = NCH * CH
SINK = NP - 1
RPW = NP // 16
RB = N // 5
NB = 8

_f32 = jnp.float32
_i32 = jnp.int32

_BCAST_DNUMS = lax.GatherDimensionNumbers(
    offset_dims=(), collapsed_slice_dims=(0,), start_index_map=(0,))


def _lane_bcast(v, r, zeros16):
    idx = jnp.reshape(zeros16 + jnp.int32(r), (16, 1))
    return lax.gather(v, idx, _BCAST_DNUMS, slice_sizes=(1,),
                      mode=lax.GatherScatterMode.PROMISE_IN_BOUNDS)


def _prep_body(x_ref, w_ref, am_ref, h_ref, s12_ref):
    xb = x_ref[...]
    hb = jnp.dot(xb, w_ref[...], preferred_element_type=_f32)
    h_ref[...] = hb
    s12_ref[...] = jnp.dot(hb, am_ref[...], preferred_element_type=_f32)


def _prep(x, W, amat):
    return pl.pallas_call(
        _prep_body,
        grid=(N // RB,),
        in_specs=[
            pl.BlockSpec((RB, DIM), lambda i: (i, 0)),
            pl.BlockSpec((DIM, NCLS), lambda i: (0, 0)),
            pl.BlockSpec((NCLS, 2), lambda i: (0, 0)),
        ],
        out_specs=[
            pl.BlockSpec((RB, NCLS), lambda i: (i, 0)),
            pl.BlockSpec((RB, 2), lambda i: (i, 0)),
        ],
        out_shape=[
            jax.ShapeDtypeStruct((N, NCLS), _f32),
            jax.ShapeDtypeStruct((N, 2), _f32),
        ],
    )(x, W, amat)


def _sc_body(h_hbm, s12_hbm, adj_hbm,
             num_out, den_out,
             srcb, dstb, s12b, rows, dvals,
             num_sh, den_sh, gsem, ssem):
    cid = lax.axis_index("c")
    sid = lax.axis_index("s")
    wid = sid * 2 + cid
    estart = wid * EW
    r0 = jnp.minimum(estart // CH, E // CH - NCH)
    off = estart - r0 * CH

    def _z(i, carry):
        for b in range(NB):
            rows[b, i, :] = jnp.zeros((16,), _f32)
        return carry
    lax.fori_loop(0, CH, _z, 0)
    for b in range(NB):
        for i in range(CH // 16):
            dvals[b, pl.ds(i * 16, 16)] = jnp.zeros((16,), _f32)
    for b in range(RPW // CH):
        pltpu.sync_copy(
            rows.at[b], num_sh.at[pl.ds(sid * RPW + b * CH, CH)])
        pltpu.sync_copy(
            dvals.at[b], den_sh.at[pl.ds(sid * RPW + b * CH, CH)])

    pltpu.sync_copy(adj_hbm.at[0, pl.ds(r0, NCH)], srcb)
    pltpu.sync_copy(adj_hbm.at[1, pl.ds(r0, NCH)], dstb)
    pltpu.sync_copy(s12_hbm, s12b)

    plsc.subcore_barrier()

    zeros16 = jnp.zeros((16,), _i32)
    ones16 = jnp.ones((16,), _i32)
    iota16 = lax.iota(_i32, 16)

    for b in range(6):
        pltpu.async_copy(h_hbm.at[dstb.at[b]], rows.at[b], gsem.at[b])

    def _oct(jj, carry):
        for b in range(NB):
            j = NB * jj + b
            rbuf = rows.at[b]
            dbuf = dvals.at[b]
            b2 = (b + 6) % NB

            pltpu.make_async_copy(
                h_hbm.at[dstb.at[j]], rbuf, gsem.at[b]).wait()

            @pl.when(j >= 2)
            def _wait_prev():
                pltpu.make_async_copy(
                    rows.at[b2], num_sh.at[srcb.at[j - 2]],
                    ssem.at[b2]).wait()
                pltpu.make_async_copy(
                    dvals.at[b2], den_sh.at[srcb.at[j - 2]],
                    ssem.at[b2]).wait()

            @pl.when(j + 6 < NCH)
            def _prefetch():
                pltpu.async_copy(
                    h_hbm.at[dstb.at[j + 6]], rows.at[b2], gsem.at[b2])

            for k in range(CH // 16):
                src16 = srcb[j, pl.ds(k * 16, 16)]
                dst16 = dstb[j, pl.ds(k * 16, 16)]
                sv = plsc.load_gather(s12b, [src16, zeros16])
                dv = plsc.load_gather(s12b, [dst16, ones16])
                t = sv + dv
                lr = jnp.where(t > 0, t, ALPHA * t)
                e = jnp.exp(-lr)
                flat = iota16 + (j * CH + k * 16)
                e = jnp.where((flat >= off) & (flat < off + EW), e, 0.0)
                dbuf[pl.ds(k * 16, 16)] = e
                for r in range(16):
                    idx = k * 16 + r
                    ev = _lane_bcast(e, r, zeros16)
                    rbuf[idx, :] = rbuf[idx, :] * ev
            pltpu.async_copy(rbuf, num_sh.at[srcb.at[j]], ssem.at[b],
                             add=True)
            pltpu.async_copy(dbuf, den_sh.at[srcb.at[j]], ssem.at[b],
                             add=True)
        return carry

    lax.fori_loop(0, NCH // NB, _oct, 0)
    for b in range(6, NB):
        j = NCH - NB + b
        pltpu.make_async_copy(
            rows.at[b], num_sh.at[srcb.at[j]], ssem.at[b]).wait()
        pltpu.make_async_copy(
            dvals.at[b], den_sh.at[srcb.at[j]], ssem.at[b]).wait()

    plsc.subcore_barrier()

    sl = pl.ds(sid * RPW, RPW)
    pltpu.sync_copy(num_sh.at[sl], num_out.at[cid, sl])
    pltpu.sync_copy(den_sh.at[sl], den_out.at[cid, sl])


def _sc_edges(h, s12, adj):
    mesh = plsc.VectorSubcoreMesh(core_axis_name="c", subcore_axis_name="s")
    f = pl.kernel(
        _sc_body,
        out_type=[
            jax.ShapeDtypeStruct((2, NP, NCLS), _f32),
            jax.ShapeDtypeStruct((2, NP), _f32),
        ],
        mesh=mesh,
        compiler_params=pltpu.CompilerParams(
            needs_layout_passes=False, use_tc_tiling_on_sc=False),
        scratch_types=[
            pltpu.VMEM((NCH, CH), _i32),
            pltpu.VMEM((NCH, CH), _i32),
            pltpu.VMEM((N, 2), _f32),
            pltpu.VMEM((NB, CH, NCLS), _f32),
            pltpu.VMEM((NB, CH), _f32),
            pltpu.VMEM_SHARED((NP, NCLS), _f32),
            pltpu.VMEM_SHARED((NP,), _f32),
            pltpu.SemaphoreType.DMA((NB,)),
            pltpu.SemaphoreType.DMA((NB,)),
        ],
    )
    return f(h, s12, adj)


def _fin_body(np_ref, dp_ref, out_ref):
    nm = np_ref[0] + np_ref[1]
    dn = dp_ref[0] + dp_ref[1] + 1e-16
    r = nm / dn
    out_ref[...] = jnp.where(r > 0, r, jnp.exp(r) - 1.0)


def _finalize(num_p, den_p3):
    return pl.pallas_call(
        _fin_body,
        grid=(N // RB,),
        in_specs=[
            pl.BlockSpec((2, RB, NCLS), lambda i: (0, i, 0)),
            pl.BlockSpec((2, RB, 1), lambda i: (0, i, 0)),
        ],
        out_specs=pl.BlockSpec((RB, NCLS), lambda i: (i, 0)),
        out_shape=jax.ShapeDtypeStruct((N, NCLS), _f32),
    )(num_p, den_p3)


def kernel(x, adj, W, a):
    amat = jnp.transpose(a.reshape(2, NCLS))
    h, s12 = _prep(x, W, amat)
    num_p, den_p = _sc_edges(h, s12, adj.reshape(2, E // CH, CH))
    return _finalize(num_p, den_p.reshape(2, NP, 1))


# speedup vs baseline: 1.5053x; 1.0039x over previous
import jax
import jax.numpy as jnp
from jax import lax
from jax.experimental import pallas as pl
from jax.experimental.pallas import tpu as pltpu
from jax.experimental.pallas import tpu_sc as plsc

N = 10000
E = 320000
DIM = 128
NCLS = 16
ALPHA = 0.2

NW = 32
NP = 10240
EW = E // NW
CH = 128
NCH = 80
RPW = NP // 16
RB = N // 5
NB = 8

_f32 = jnp.float32
_i32 = jnp.int32

_BCAST_DNUMS = lax.GatherDimensionNumbers(
    offset_dims=(), collapsed_slice_dims=(0,), start_index_map=(0,))


def _lane_bcast(v, r, zeros16):
    idx = jnp.reshape(zeros16 + jnp.int32(r), (16, 1))
    return lax.gather(v, idx, _BCAST_DNUMS, slice_sizes=(1,),
                      mode=lax.GatherScatterMode.PROMISE_IN_BOUNDS)


def _prep_body(x_ref, w_ref, am_ref, h_ref, s12_ref):
    xb = x_ref[...]
    hb = jnp.dot(xb, w_ref[...], preferred_element_type=_f32)
    h_ref[...] = hb
    s12_ref[...] = jnp.dot(hb, am_ref[...], preferred_element_type=_f32)


def _prep(x, W, amat):
    return pl.pallas_call(
        _prep_body,
        grid=(N // RB,),
        in_specs=[
            pl.BlockSpec((RB, DIM), lambda i: (i, 0)),
            pl.BlockSpec((DIM, NCLS), lambda i: (0, 0)),
            pl.BlockSpec((NCLS, 2), lambda i: (0, 0)),
        ],
        out_specs=[
            pl.BlockSpec((RB, NCLS), lambda i: (i, 0)),
            pl.BlockSpec((RB, 2), lambda i: (i, 0)),
        ],
        out_shape=[
            jax.ShapeDtypeStruct((N, NCLS), _f32),
            jax.ShapeDtypeStruct((N, 2), _f32),
        ],
    )(x, W, amat)


def _sc_body(h_hbm, s12_hbm, adj_hbm,
             num_out, den_out,
             srcb, dstb, s12b, rows, dvals,
             num_sh, den_sh, gsem, ssem):
    cid = lax.axis_index("c")
    sid = lax.axis_index("s")
    wid = sid * 2 + cid
    estart = wid * EW
    r0 = jnp.minimum(estart // CH, E // CH - NCH)
    off = estart - r0 * CH

    def _z(i, carry):
        for b in range(NB):
            rows[b, i, :] = jnp.zeros((16,), _f32)
        return carry
    lax.fori_loop(0, CH, _z, 0)
    for b in range(NB):
        for i in range(CH // 16):
            dvals[b, pl.ds(i * 16, 16)] = jnp.zeros((16,), _f32)
    for b in range(RPW // CH):
        pltpu.sync_copy(
            rows.at[b], num_sh.at[pl.ds(sid * RPW + b * CH, CH)])
        pltpu.sync_copy(
            dvals.at[b], den_sh.at[pl.ds(sid * RPW + b * CH, CH)])

    pltpu.sync_copy(adj_hbm.at[0, pl.ds(r0, NCH)], srcb)
    pltpu.sync_copy(adj_hbm.at[1, pl.ds(r0, NCH)], dstb)
    pltpu.sync_copy(s12_hbm, s12b)

    plsc.subcore_barrier()

    zeros16 = jnp.zeros((16,), _i32)
    ones16 = jnp.ones((16,), _i32)
    iota16 = lax.iota(_i32, 16)

    for b in range(6):
        pltpu.async_copy(h_hbm.at[dstb.at[b]], rows.at[b], gsem.at[b])

    def _oct(jj, carry):
        for b in range(NB):
            j = NB * jj + b
            rbuf = rows.at[b]
            dbuf = dvals.at[b]
            b2 = (b + 6) % NB

            pltpu.make_async_copy(
                h_hbm.at[dstb.at[j]], rbuf, gsem.at[b]).wait()

            @pl.when(j >= 2)
            def _wait_prev():
                pltpu.make_async_copy(
                    rows.at[b2], num_sh.at[srcb.at[j - 2]],
                    ssem.at[b2]).wait()
                pltpu.make_async_copy(
                    dvals.at[b2], den_sh.at[srcb.at[j - 2]],
                    ssem.at[b2]).wait()

            @pl.when(j + 6 < NCH)
            def _prefetch():
                pltpu.async_copy(
                    h_hbm.at[dstb.at[j + 6]], rows.at[b2], gsem.at[b2])

            for k in range(CH // 16):
                src16 = srcb[j, pl.ds(k * 16, 16)]
                dst16 = dstb[j, pl.ds(k * 16, 16)]
                sv = plsc.load_gather(s12b, [src16, zeros16])
                dv = plsc.load_gather(s12b, [dst16, ones16])
                t = sv + dv
                lr = jnp.where(t > 0, t, ALPHA * t)
                e = jnp.exp(-lr)
                flat = iota16 + (j * CH + k * 16)
                e = jnp.where((flat >= off) & (flat < off + EW), e, 0.0)
                dbuf[pl.ds(k * 16, 16)] = e
                for r in range(16):
                    idx = k * 16 + r
                    ev = _lane_bcast(e, r, zeros16)
                    rbuf[idx, :] = rbuf[idx, :] * ev
            pltpu.async_copy(rbuf, num_sh.at[srcb.at[j]], ssem.at[b],
                             add=True)
            pltpu.async_copy(dbuf, den_sh.at[srcb.at[j]], ssem.at[b],
                             add=True)
        return carry

    lax.fori_loop(0, NCH // NB, _oct, 0)
    for b in range(6, NB):
        j = NCH - NB + b
        pltpu.make_async_copy(
            rows.at[b], num_sh.at[srcb.at[j]], ssem.at[b]).wait()
        pltpu.make_async_copy(
            dvals.at[b], den_sh.at[srcb.at[j]], ssem.at[b]).wait()

    plsc.subcore_barrier()

    sl = pl.ds(sid * RPW, RPW)
    pltpu.sync_copy(num_sh.at[sl], num_out.at[cid, sl])
    pltpu.sync_copy(den_sh.at[sl], den_out.at[cid, sl])


def _sc_edges(h, s12, adj):
    mesh = plsc.VectorSubcoreMesh(core_axis_name="c", subcore_axis_name="s")
    f = pl.kernel(
        _sc_body,
        out_type=[
            jax.ShapeDtypeStruct((2, NP, NCLS), _f32),
            jax.ShapeDtypeStruct((2, NP), _f32),
        ],
        mesh=mesh,
        compiler_params=pltpu.CompilerParams(
            needs_layout_passes=False, use_tc_tiling_on_sc=False),
        scratch_types=[
            pltpu.VMEM((NCH, CH), _i32),
            pltpu.VMEM((NCH, CH), _i32),
            pltpu.VMEM((N, 2), _f32),
            pltpu.VMEM((NB, CH, NCLS), _f32),
            pltpu.VMEM((NB, CH), _f32),
            pltpu.VMEM_SHARED((NP, NCLS), _f32),
            pltpu.VMEM_SHARED((NP,), _f32),
            pltpu.SemaphoreType.DMA((NB,)),
            pltpu.SemaphoreType.DMA((NB,)),
        ],
    )
    return f(h, s12, adj)


def _fin_body(np_ref, dp_ref, out_ref):
    nm = np_ref[0] + np_ref[1]
    dn = dp_ref[0] + dp_ref[1] + 1e-16
    r = nm / dn
    out_ref[...] = jnp.where(r > 0, r, jnp.exp(r) - 1.0)


def _finalize(num_p, den_p3):
    return pl.pallas_call(
        _fin_body,
        grid=(N // RB,),
        in_specs=[
            pl.BlockSpec((2, RB, NCLS), lambda i: (0, i, 0)),
            pl.BlockSpec((2, RB, 1), lambda i: (0, i, 0)),
        ],
        out_specs=pl.BlockSpec((RB, NCLS), lambda i: (i, 0)),
        out_shape=jax.ShapeDtypeStruct((N, NCLS), _f32),
    )(num_p, den_p3)


def kernel(x, adj, W, a):
    amat = jnp.transpose(a.reshape(2, NCLS))
    h, s12 = _prep(x, W, amat)
    num_p, den_p = _sc_edges(h, s12, adj.reshape(2, E // CH, CH))
    return _finalize(num_p, den_p.reshape(2, NP, 1))
